# baseline (speedup 1.0000x reference)
import functools

import jax
import jax.numpy as jnp
from jax import lax
from jax.experimental import pallas as pl
from jax.experimental.pallas import tpu as pltpu
from jax.experimental.pallas import tpu_sc as plsc

TOPK2 = 2
BLK = 256
BH = 512


def _sc_gather_rows(table, idx, n_rows, d):
    info = plsc.get_sparse_core_info()
    nc, ns = info.num_cores, info.num_subcores
    nw = nc * ns
    b_per_w = n_rows // nw
    ch = 64
    n_ch = b_per_w // ch
    mesh = plsc.VectorSubcoreMesh(core_axis_name="c", subcore_axis_name="s")

    @functools.partial(
        pl.kernel,
        out_type=jax.ShapeDtypeStruct((n_rows, d), jnp.float32),
        mesh=mesh,
        scratch_types=[
            pltpu.VMEM((ch,), jnp.int32),
            pltpu.VMEM((ch, d), jnp.float32),
            pltpu.SemaphoreType.DMA,
        ],
    )
    def gk(table_hbm, idx_hbm, out_hbm, idx_v, rows_v, sem):
        wid = lax.axis_index("s") * nc + lax.axis_index("c")
        base = wid * b_per_w

        def body(i, carry):
            off = pl.multiple_of(base + i * ch, 8)
            pltpu.sync_copy(idx_hbm.at[pl.ds(off, ch)], idx_v)
            pltpu.async_copy(table_hbm.at[idx_v], rows_v, sem).wait()
            pltpu.sync_copy(rows_v, out_hbm.at[pl.ds(off, ch)])
            return carry

        lax.fori_loop(0, n_ch, body, 0)

    return gk(table, idx)


def _gemm_body(be_ref, nv_ref, xs_ref, w1_ref, b1_ref, w2_ref, b2_ref,
               ys_ref, yacc):
    m = pl.program_id(0)
    h = pl.program_id(1)
    nh = pl.num_programs(1)

    @pl.when(m < nv_ref[0])
    def _():
        @pl.when(h == 0)
        def _():
            yacc[...] = jnp.broadcast_to(b2_ref[0, 0], yacc.shape)

        xb = xs_ref[...].astype(jnp.bfloat16)
        hd = lax.dot_general(xb, w1_ref[0], (((1,), (1,)), ((), ())),
                             preferred_element_type=jnp.float32)
        hd = hd + b1_ref[0, 0]
        act = 0.5 * hd * (1.0 + lax.erf(hd * 0.7071067811865476))
        yacc[...] += lax.dot_general(act.astype(jnp.bfloat16), w2_ref[0],
                                     (((1,), (1,)), ((), ())),
                                     preferred_element_type=jnp.float32)

        @pl.when(h == nh - 1)
        def _():
            ys_ref[...] = yacc[...]


def _grouped_ffn(xs, w1, b1, w2, b2, block_expert, nvalid, npad):
    e, hdim, d = w1.shape
    nblk = npad // BLK
    nh = hdim // BH
    grid_spec = pltpu.PrefetchScalarGridSpec(
        num_scalar_prefetch=2,
        grid=(nblk, nh),
        in_specs=[
            pl.BlockSpec((BLK, d), lambda m, h, be, nv: (m, 0)),
            pl.BlockSpec((1, BH, d), lambda m, h, be, nv: (be[m], h, 0)),
            pl.BlockSpec((1, 1, BH), lambda m, h, be, nv: (be[m], 0, h)),
            pl.BlockSpec((1, d, BH), lambda m, h, be, nv: (be[m], 0, h)),
            pl.BlockSpec((1, 1, d), lambda m, h, be, nv: (be[m], 0, 0)),
        ],
        out_specs=pl.BlockSpec((BLK, d), lambda m, h, be, nv: (m, 0)),
        scratch_shapes=[pltpu.VMEM((BLK, d), jnp.float32)],
    )
    return pl.pallas_call(
        _gemm_body,
        grid_spec=grid_spec,
        out_shape=jax.ShapeDtypeStruct((npad, d), jnp.float32),
    )(block_expert, nvalid, xs, w1.astype(jnp.bfloat16),
      b1.reshape(e, 1, hdim), w2.astype(jnp.bfloat16), b2.reshape(e, 1, d))


def _combine_body(ya_ref, yb_ref, ga_ref, gb_ref, out_ref):
    out_ref[...] = ya_ref[...] * ga_ref[0] + yb_ref[...] * gb_ref[0]


def _combine(ysg, ga, gb, n, d):
    nt = n // BLK
    return pl.pallas_call(
        _combine_body,
        grid=(nt,),
        in_specs=[
            pl.BlockSpec((BLK, d), lambda i: (i, 0)),
            pl.BlockSpec((BLK, d), lambda i: (nt + i, 0)),
            pl.BlockSpec((1, BLK, 1), lambda i: (i, 0, 0)),
            pl.BlockSpec((1, BLK, 1), lambda i: (i, 0, 0)),
        ],
        out_specs=pl.BlockSpec((BLK, d), lambda i: (i, 0)),
        out_shape=jax.ShapeDtypeStruct((n, d), jnp.float32),
    )(ysg, ysg, ga, gb)


def kernel(x, Wr, W1, b1, W2, b2):
    b, s, d = x.shape
    e = Wr.shape[0]
    hdim = W1.shape[1]
    n = b * s
    ns = n * TOPK2
    npad = ns + e * BLK
    nblk = npad // BLK

    xf = x.reshape(n, d)

    logits = xf @ Wr.T
    probs = jax.nn.softmax(logits, axis=-1)
    topv, topi = lax.top_k(logits, TOPK2)
    topp = jax.nn.softmax(topv, axis=-1)
    onehot = jax.nn.one_hot(topi, e, dtype=jnp.float32)
    f_i = onehot.mean(axis=(0, 1))
    p_i = probs.mean(axis=0)
    aux_loss = e * jnp.sum(f_i * p_i)

    slot_e = topi.reshape(-1)
    slot_g = topp.reshape(-1)
    slot_t = jnp.arange(ns, dtype=jnp.int32) // TOPK2
    inc = (slot_e[:, None] == jnp.arange(e)[None, :]).astype(jnp.int32)
    csum = jnp.cumsum(inc, axis=0)
    rank = jnp.take_along_axis(csum, slot_e[:, None], axis=1)[:, 0] - 1
    counts = csum[-1]
    pcounts = ((counts + BLK - 1) // BLK) * BLK
    ends = jnp.cumsum(pcounts)
    base = ends - pcounts
    pos = (base[slot_e] + rank).astype(jnp.int32)
    nvalid = (ends[-1] // BLK).astype(jnp.int32)[None]
    rows = (jnp.arange(npad, dtype=jnp.int32) % n).at[pos].set(slot_t)
    block_expert = jnp.minimum(
        jnp.searchsorted(ends, jnp.arange(nblk) * BLK, side="right"),
        e - 1).astype(jnp.int32)

    xs = _sc_gather_rows(xf, rows, npad, d)

    ys = _grouped_ffn(xs, W1, b1, W2, b2, block_expert, nvalid, npad)

    idx2 = jnp.concatenate([pos[0::2], pos[1::2]])
    ysg = _sc_gather_rows(ys, idx2, ns, d)
    nt = n // BLK
    ga = slot_g[0::2].reshape(nt, BLK, 1)
    gb = slot_g[1::2].reshape(nt, BLK, 1)
    out = _combine(ysg, ga, gb, n, d)

    return out.reshape(b, s, d), aux_loss


# speedup vs baseline: 1.0492x; 1.0492x over previous
import functools

import jax
import jax.numpy as jnp
from jax import lax
from jax.experimental import pallas as pl
from jax.experimental.pallas import tpu as pltpu
from jax.experimental.pallas import tpu_sc as plsc

TOPK2 = 2
BLK = 256
BH = 512


def _sc_gather_rows(table, idx, n_rows, d):
    info = plsc.get_sparse_core_info()
    nc, ns = info.num_cores, info.num_subcores
    nw = nc * ns
    b_per_w = n_rows // nw
    ch = 64
    n_ch = b_per_w // ch
    mesh = plsc.VectorSubcoreMesh(core_axis_name="c", subcore_axis_name="s")

    @functools.partial(
        pl.kernel,
        out_type=jax.ShapeDtypeStruct((n_rows, d), jnp.float32),
        mesh=mesh,
        scratch_types=[
            pltpu.VMEM((ch,), jnp.int32),
            pltpu.VMEM((ch, d), jnp.float32),
            pltpu.SemaphoreType.DMA,
        ],
    )
    def gk(table_hbm, idx_hbm, out_hbm, idx_v, rows_v, sem):
        wid = lax.axis_index("s") * nc + lax.axis_index("c")
        base = wid * b_per_w

        def body(i, carry):
            off = pl.multiple_of(base + i * ch, 8)
            pltpu.sync_copy(idx_hbm.at[pl.ds(off, ch)], idx_v)
            pltpu.async_copy(table_hbm.at[idx_v], rows_v, sem).wait()
            pltpu.sync_copy(rows_v, out_hbm.at[pl.ds(off, ch)])
            return carry

        lax.fori_loop(0, n_ch, body, 0)

    return gk(table, idx)


def _gemm_body(be_ref, nv_ref, xs_ref, w1_ref, b1_ref, w2_ref, b2_ref,
               ys_ref, yacc):
    m = pl.program_id(0)
    h = pl.program_id(1)
    nh = pl.num_programs(1)

    @pl.when(m < nv_ref[0])
    def _():
        @pl.when(h == 0)
        def _():
            yacc[...] = jnp.broadcast_to(b2_ref[0, 0], yacc.shape)

        xb = xs_ref[...]
        hd = lax.dot_general(xb, w1_ref[0], (((1,), (1,)), ((), ())),
                             preferred_element_type=jnp.float32)
        hd = hd + b1_ref[0, 0]
        act = 0.5 * hd * (1.0 + lax.erf(hd * 0.7071067811865476))
        yacc[...] += lax.dot_general(act, w2_ref[0], (((1,), (1,)), ((), ())),
                                     preferred_element_type=jnp.float32)

        @pl.when(h == nh - 1)
        def _():
            ys_ref[...] = yacc[...]


def _grouped_ffn(xs, w1, b1, w2, b2, block_expert, nvalid, npad):
    e, hdim, d = w1.shape
    nblk = npad // BLK
    nh = hdim // BH
    grid_spec = pltpu.PrefetchScalarGridSpec(
        num_scalar_prefetch=2,
        grid=(nblk, nh),
        in_specs=[
            pl.BlockSpec((BLK, d), lambda m, h, be, nv: (m, 0)),
            pl.BlockSpec((1, BH, d), lambda m, h, be, nv: (be[m], h, 0)),
            pl.BlockSpec((1, 1, BH), lambda m, h, be, nv: (be[m], 0, h)),
            pl.BlockSpec((1, d, BH), lambda m, h, be, nv: (be[m], 0, h)),
            pl.BlockSpec((1, 1, d), lambda m, h, be, nv: (be[m], 0, 0)),
        ],
        out_specs=pl.BlockSpec((BLK, d), lambda m, h, be, nv: (m, 0)),
        scratch_shapes=[pltpu.VMEM((BLK, d), jnp.float32)],
    )
    return pl.pallas_call(
        _gemm_body,
        grid_spec=grid_spec,
        out_shape=jax.ShapeDtypeStruct((npad, d), jnp.float32),
    )(block_expert, nvalid, xs, w1, b1.reshape(e, 1, hdim), w2,
      b2.reshape(e, 1, d))


def _combine_body(ya_ref, yb_ref, ga_ref, gb_ref, out_ref):
    out_ref[...] = ya_ref[...] * ga_ref[0] + yb_ref[...] * gb_ref[0]


def _combine(ysg, ga, gb, n, d):
    nt = n // BLK
    return pl.pallas_call(
        _combine_body,
        grid=(nt,),
        in_specs=[
            pl.BlockSpec((BLK, d), lambda i: (i, 0)),
            pl.BlockSpec((BLK, d), lambda i: (nt + i, 0)),
            pl.BlockSpec((1, BLK, 1), lambda i: (i, 0, 0)),
            pl.BlockSpec((1, BLK, 1), lambda i: (i, 0, 0)),
        ],
        out_specs=pl.BlockSpec((BLK, d), lambda i: (i, 0)),
        out_shape=jax.ShapeDtypeStruct((n, d), jnp.float32),
    )(ysg, ysg, ga, gb)


def kernel(x, Wr, W1, b1, W2, b2):
    b, s, d = x.shape
    e = Wr.shape[0]
    hdim = W1.shape[1]
    n = b * s
    ns = n * TOPK2
    npad = ns + e * BLK
    nblk = npad // BLK

    xf = x.reshape(n, d)

    logits = xf @ Wr.T
    probs = jax.nn.softmax(logits, axis=-1)
    topv, topi = lax.top_k(logits, TOPK2)
    topp = jax.nn.softmax(topv, axis=-1)
    onehot = jax.nn.one_hot(topi, e, dtype=jnp.float32)
    f_i = onehot.mean(axis=(0, 1))
    p_i = probs.mean(axis=0)
    aux_loss = e * jnp.sum(f_i * p_i)

    slot_e = topi.reshape(-1)
    slot_g = topp.reshape(-1)
    slot_t = jnp.arange(ns, dtype=jnp.int32) // TOPK2
    inc = (slot_e[:, None] == jnp.arange(e)[None, :]).astype(jnp.int32)
    csum = jnp.cumsum(inc, axis=0)
    rank = jnp.take_along_axis(csum, slot_e[:, None], axis=1)[:, 0] - 1
    counts = csum[-1]
    pcounts = ((counts + BLK - 1) // BLK) * BLK
    ends = jnp.cumsum(pcounts)
    base = ends - pcounts
    pos = (base[slot_e] + rank).astype(jnp.int32)
    nvalid = (ends[-1] // BLK).astype(jnp.int32)[None]
    rows = (jnp.arange(npad, dtype=jnp.int32) % n).at[pos].set(slot_t)
    block_expert = jnp.minimum(
        jnp.searchsorted(ends, jnp.arange(nblk) * BLK, side="right"),
        e - 1).astype(jnp.int32)

    xs = _sc_gather_rows(xf, rows, npad, d)

    ys = _grouped_ffn(xs, W1, b1, W2, b2, block_expert, nvalid, npad)

    idx2 = jnp.concatenate([pos[0::2], pos[1::2]])
    ysg = _sc_gather_rows(ys, idx2, ns, d)
    nt = n // BLK
    ga = slot_g[0::2].reshape(nt, BLK, 1)
    gb = slot_g[1::2].reshape(nt, BLK, 1)
    out = _combine(ysg, ga, gb, n, d)

    return out.reshape(b, s, d), aux_loss


# speedup vs baseline: 1.1704x; 1.1154x over previous
import functools

import jax
import jax.numpy as jnp
from jax import lax
from jax.experimental import pallas as pl
from jax.experimental.pallas import tpu as pltpu
from jax.experimental.pallas import tpu_sc as plsc

TOPK2 = 2
BLK = 256
BH = 512


def _sc_gather_rows(table, idx, n_rows, d):
    info = plsc.get_sparse_core_info()
    nc, ns = info.num_cores, info.num_subcores
    nw = nc * ns
    b_per_w = n_rows // nw
    ch = 64
    n_ch = b_per_w // ch
    mesh = plsc.VectorSubcoreMesh(core_axis_name="c", subcore_axis_name="s")

    @functools.partial(
        pl.kernel,
        out_type=jax.ShapeDtypeStruct((n_rows, d), jnp.float32),
        mesh=mesh,
        scratch_types=[
            pltpu.VMEM((ch,), jnp.int32),
            pltpu.VMEM((ch, d), jnp.float32),
            pltpu.SemaphoreType.DMA,
        ],
    )
    def gk(table_hbm, idx_hbm, out_hbm, idx_v, rows_v, sem):
        wid = lax.axis_index("s") * nc + lax.axis_index("c")
        base = wid * b_per_w

        def body(i, carry):
            off = pl.multiple_of(base + i * ch, 8)
            pltpu.sync_copy(idx_hbm.at[pl.ds(off, ch)], idx_v)
            pltpu.async_copy(table_hbm.at[idx_v], rows_v, sem).wait()
            pltpu.sync_copy(rows_v, out_hbm.at[pl.ds(off, ch)])
            return carry

        lax.fori_loop(0, n_ch, body, 0)

    return gk(table, idx)


def _gemm_body(be_ref, nv_ref, xs_ref, w1_ref, b1_ref, w2_ref, b2_ref,
               ys_ref):
    h = pl.program_id(0)
    m = pl.program_id(1)

    @pl.when(m < nv_ref[0])
    def _():
        xb = xs_ref[...]
        hd = lax.dot_general(xb, w1_ref[0], (((1,), (1,)), ((), ())),
                             preferred_element_type=jnp.float32)
        hd = hd + b1_ref[0, 0]
        act = 0.5 * hd * (1.0 + lax.erf(hd * 0.7071067811865476))
        yp = lax.dot_general(act, w2_ref[0], (((1,), (1,)), ((), ())),
                             preferred_element_type=jnp.float32)
        @pl.when(h == 0)
        def _():
            ys_ref[pl.ds(m * BLK, BLK), :] = (
                yp + jnp.broadcast_to(b2_ref[0, 0], yp.shape))

        @pl.when(h != 0)
        def _():
            ys_ref[pl.ds(m * BLK, BLK), :] += yp


def _grouped_ffn(xs, w1, b1, w2, b2, block_expert, nvalid, npad):
    e, hdim, d = w1.shape
    nblk = npad // BLK
    nh = hdim // BH
    grid_spec = pltpu.PrefetchScalarGridSpec(
        num_scalar_prefetch=2,
        grid=(nh, nblk),
        in_specs=[
            pl.BlockSpec((BLK, d), lambda h, m, be, nv: (m, 0)),
            pl.BlockSpec((1, BH, d), lambda h, m, be, nv: (be[m], h, 0)),
            pl.BlockSpec((1, 1, BH), lambda h, m, be, nv: (be[m], 0, h)),
            pl.BlockSpec((1, d, BH), lambda h, m, be, nv: (be[m], 0, h)),
            pl.BlockSpec((1, 1, d), lambda h, m, be, nv: (be[m], 0, 0)),
        ],
        out_specs=pl.BlockSpec((npad, d), lambda h, m, be, nv: (0, 0)),
    )
    return pl.pallas_call(
        _gemm_body,
        grid_spec=grid_spec,
        out_shape=jax.ShapeDtypeStruct((npad, d), jnp.float32),
    )(block_expert, nvalid, xs, w1, b1.reshape(e, 1, hdim), w2,
      b2.reshape(e, 1, d))


def _combine_body(ya_ref, yb_ref, ga_ref, gb_ref, out_ref):
    out_ref[...] = ya_ref[...] * ga_ref[0] + yb_ref[...] * gb_ref[0]


def _combine(ysg, ga, gb, n, d):
    nt = n // BLK
    return pl.pallas_call(
        _combine_body,
        grid=(nt,),
        in_specs=[
            pl.BlockSpec((BLK, d), lambda i: (i, 0)),
            pl.BlockSpec((BLK, d), lambda i: (nt + i, 0)),
            pl.BlockSpec((1, BLK, 1), lambda i: (i, 0, 0)),
            pl.BlockSpec((1, BLK, 1), lambda i: (i, 0, 0)),
        ],
        out_specs=pl.BlockSpec((BLK, d), lambda i: (i, 0)),
        out_shape=jax.ShapeDtypeStruct((n, d), jnp.float32),
    )(ysg, ysg, ga, gb)


def kernel(x, Wr, W1, b1, W2, b2):
    b, s, d = x.shape
    e = Wr.shape[0]
    hdim = W1.shape[1]
    n = b * s
    ns = n * TOPK2
    npad = ns + e * BLK
    nblk = npad // BLK

    xf = x.reshape(n, d)

    logits = xf @ Wr.T
    probs = jax.nn.softmax(logits, axis=-1)
    topv, topi = lax.top_k(logits, TOPK2)
    topp = jax.nn.softmax(topv, axis=-1)
    onehot = jax.nn.one_hot(topi, e, dtype=jnp.float32)
    f_i = onehot.mean(axis=(0, 1))
    p_i = probs.mean(axis=0)
    aux_loss = e * jnp.sum(f_i * p_i)

    slot_e = topi.reshape(-1)
    slot_g = topp.reshape(-1)
    slot_t = jnp.arange(ns, dtype=jnp.int32) // TOPK2
    inc = (slot_e[:, None] == jnp.arange(e)[None, :]).astype(jnp.int32)
    csum = jnp.cumsum(inc, axis=0)
    rank = jnp.take_along_axis(csum, slot_e[:, None], axis=1)[:, 0] - 1
    counts = csum[-1]
    pcounts = ((counts + BLK - 1) // BLK) * BLK
    ends = jnp.cumsum(pcounts)
    base = ends - pcounts
    pos = (base[slot_e] + rank).astype(jnp.int32)
    nvalid = (ends[-1] // BLK).astype(jnp.int32)[None]
    rows = (jnp.arange(npad, dtype=jnp.int32) % n).at[pos].set(slot_t)
    block_expert = jnp.minimum(
        jnp.searchsorted(ends, jnp.arange(nblk) * BLK, side="right"),
        e - 1).astype(jnp.int32)

    xs = _sc_gather_rows(xf, rows, npad, d)

    ys = _grouped_ffn(xs, W1, b1, W2, b2, block_expert, nvalid, npad)

    idx2 = jnp.concatenate([pos[0::2], pos[1::2]])
    ysg = _sc_gather_rows(ys, idx2, ns, d)
    nt = n // BLK
    ga = slot_g[0::2].reshape(nt, BLK, 1)
    gb = slot_g[1::2].reshape(nt, BLK, 1)
    out = _combine(ysg, ga, gb, n, d)

    return out.reshape(b, s, d), aux_loss
